# Initial kernel scaffold; baseline (speedup 1.0000x reference)
import jax
import jax.numpy as jnp
from jax.experimental import pallas as pl


def kernel(x, edge_index, edge_attr, W1, b1, W2, b2, Wp1, bp1, Wp2, bp2):
    raise NotImplementedError("write your pallas kernel here")


# speedup vs baseline: 5.7618x; 5.7618x over previous
import functools

import jax
import jax.numpy as jnp
from jax import lax
from jax.experimental import pallas as pl
from jax.experimental.pallas import tpu as pltpu
from jax.experimental.pallas import tpu_sc as plsc

N_NODES = 10000
N_EDGES = 320000
D = 128
D_EDGE = 16

NC = 2
NS = 16
NW = NC * NS
CH = 128
NCH = 79
EPT = NCH * CH
EPAD = NW * EPT
NPAD = 10112
ZROWS = NPAD // NS
OROWS = NPAD // NS

_mesh = plsc.VectorSubcoreMesh(core_axis_name="c", subcore_axis_name="s")


def _deg_body(dstp, ones_h, z128, dout, idx_v, ones_v, acc):
    c = lax.axis_index("c")
    s = lax.axis_index("s")
    wid = c * NS + s
    pltpu.sync_copy(z128, acc.at[pl.ds(s * ZROWS, ZROWS)])
    pltpu.sync_copy(ones_h, ones_v)
    pltpu.sync_copy(dstp.at[wid], idx_v)
    plsc.subcore_barrier()

    def body(j, carry):
        pltpu.sync_copy(ones_v, acc.at[idx_v.at[j]], add=True)
        return carry

    lax.fori_loop(0, NCH, body, 0)
    plsc.subcore_barrier()
    sl = pl.ds(s * OROWS, OROWS)
    pltpu.sync_copy(acc.at[sl], dout.at[c].at[sl])


_deg_call = pl.kernel(
    _deg_body,
    out_type=jax.ShapeDtypeStruct((NC, NPAD, D), jnp.float32),
    mesh=_mesh,
    scratch_types=[
        pltpu.VMEM((NCH, CH), jnp.int32),
        pltpu.VMEM((CH, D), jnp.float32),
        pltpu.VMEM_SHARED((NPAD, D), jnp.float32),
    ],
)


def _agg_body(g_h, srcp, dstp, z128, out, sidx, didx, rows, acc):
    c = lax.axis_index("c")
    s = lax.axis_index("s")
    wid = c * NS + s
    pltpu.sync_copy(z128, acc.at[pl.ds(s * ZROWS, ZROWS)])
    pltpu.sync_copy(srcp.at[wid], sidx)
    pltpu.sync_copy(dstp.at[wid], didx)
    plsc.subcore_barrier()

    def body(j, carry):
        pltpu.sync_copy(g_h.at[sidx.at[j]], rows)
        pltpu.sync_copy(rows, acc.at[didx.at[j]], add=True)
        return carry

    lax.fori_loop(0, NCH, body, 0)
    plsc.subcore_barrier()
    sl = pl.ds(s * OROWS, OROWS)
    pltpu.sync_copy(acc.at[sl], out.at[c].at[sl])


_agg_call = pl.kernel(
    _agg_body,
    out_type=jax.ShapeDtypeStruct((NC, NPAD, D), jnp.float32),
    mesh=_mesh,
    scratch_types=[
        pltpu.VMEM((NCH, CH), jnp.int32),
        pltpu.VMEM((NCH, CH), jnp.int32),
        pltpu.VMEM((CH, D), jnp.float32),
        pltpu.VMEM_SHARED((NPAD, D), jnp.float32),
    ],
)


def _egather_body(a_h, b_h, srcp, dstp, ga, gb, sidx, didx, bufa, bufb,
                  sema, semb):
    c = lax.axis_index("c")
    s = lax.axis_index("s")
    wid = c * NS + s
    pltpu.sync_copy(srcp.at[wid], sidx)
    pltpu.sync_copy(dstp.at[wid], didx)

    def body(j, carry):
        ca = pltpu.async_copy(a_h.at[sidx.at[j]], bufa, sema)
        cb = pltpu.async_copy(b_h.at[didx.at[j]], bufb, semb)
        ca.wait()
        cb.wait()
        base = wid * EPT + j * CH
        pltpu.sync_copy(bufa, ga.at[pl.ds(base, CH)])
        pltpu.sync_copy(bufb, gb.at[pl.ds(base, CH)])
        return carry

    lax.fori_loop(0, NCH, body, 0)


_egather_call = pl.kernel(
    _egather_body,
    out_type=(
        jax.ShapeDtypeStruct((EPAD, D), jnp.float32),
        jax.ShapeDtypeStruct((EPAD, D), jnp.float32),
    ),
    mesh=_mesh,
    scratch_types=[
        pltpu.VMEM((NCH, CH), jnp.int32),
        pltpu.VMEM((NCH, CH), jnp.int32),
        pltpu.VMEM((CH, D), jnp.float32),
        pltpu.VMEM((CH, D), jnp.float32),
        pltpu.SemaphoreType.DMA,
        pltpu.SemaphoreType.DMA,
    ],
)


BN = 2000
BE = 512


def _tc1_body(x_ref, w1_ref, d0_ref, d1_ref, g1_ref, dinv_ref):
    deg = 1.0 + d0_ref[0, :, 0:1] + d1_ref[0, :, 0:1]
    dinv = lax.rsqrt(deg)
    h = jnp.dot(x_ref[...], w1_ref[...], preferred_element_type=jnp.float32)
    g1_ref[...] = h * dinv
    dinv_ref[...] = jnp.broadcast_to(dinv, (BN, 16))


def _tc1(x, w1, dpart):
    return pl.pallas_call(
        _tc1_body,
        grid=(N_NODES // BN,),
        in_specs=[
            pl.BlockSpec((BN, D), lambda i: (i, 0)),
            pl.BlockSpec((D, D), lambda i: (0, 0)),
            pl.BlockSpec((1, BN, D), lambda i: (0, i, 0)),
            pl.BlockSpec((1, BN, D), lambda i: (1, i, 0)),
        ],
        out_specs=[
            pl.BlockSpec((BN, D), lambda i: (i, 0)),
            pl.BlockSpec((BN, 16), lambda i: (i, 0)),
        ],
        out_shape=[
            jax.ShapeDtypeStruct((N_NODES, D), jnp.float32),
            jax.ShapeDtypeStruct((N_NODES, 16), jnp.float32),
        ],
    )(x, w1, dpart, dpart)


def _tc2_body(sa_ref, sb_ref, g1_ref, dinv_ref, b1_ref, w2_ref, g2_ref):
    dinv = dinv_ref[:, 0:1]
    x1 = jax.nn.relu(dinv * (sa_ref[0] + sb_ref[0] + g1_ref[...])
                     + b1_ref[...][None, :])
    g2_ref[...] = jnp.dot(x1, w2_ref[...],
                          preferred_element_type=jnp.float32) * dinv


def _tc2(s1, g1, dinv, b1, w2):
    return pl.pallas_call(
        _tc2_body,
        grid=(N_NODES // BN,),
        in_specs=[
            pl.BlockSpec((1, BN, D), lambda i: (0, i, 0)),
            pl.BlockSpec((1, BN, D), lambda i: (1, i, 0)),
            pl.BlockSpec((BN, D), lambda i: (i, 0)),
            pl.BlockSpec((BN, 16), lambda i: (i, 0)),
            pl.BlockSpec((D,), lambda i: (0,)),
            pl.BlockSpec((D, D), lambda i: (0, 0)),
        ],
        out_specs=pl.BlockSpec((BN, D), lambda i: (i, 0)),
        out_shape=jax.ShapeDtypeStruct((N_NODES, D), jnp.float32),
    )(s1, s1, g1, dinv, b1, w2)


def _tc3_body(sa_ref, sb_ref, g2_ref, dinv_ref, b2_ref, wp1_ref,
              a_ref, b_ref):
    dinv = dinv_ref[:, 0:1]
    x2 = dinv * (sa_ref[0] + sb_ref[0] + g2_ref[...]) + b2_ref[...][None, :]
    a_ref[...] = jnp.dot(x2, wp1_ref[0:D, :],
                         preferred_element_type=jnp.float32)
    b_ref[...] = jnp.dot(x2, wp1_ref[D:2 * D, :],
                         preferred_element_type=jnp.float32)


def _tc3(s2, g2, dinv, b2, wp1):
    return pl.pallas_call(
        _tc3_body,
        grid=(N_NODES // BN,),
        in_specs=[
            pl.BlockSpec((1, BN, D), lambda i: (0, i, 0)),
            pl.BlockSpec((1, BN, D), lambda i: (1, i, 0)),
            pl.BlockSpec((BN, D), lambda i: (i, 0)),
            pl.BlockSpec((BN, 16), lambda i: (i, 0)),
            pl.BlockSpec((D,), lambda i: (0,)),
            pl.BlockSpec((2 * D + D_EDGE, D), lambda i: (0, 0)),
        ],
        out_specs=[
            pl.BlockSpec((BN, D), lambda i: (i, 0)),
            pl.BlockSpec((BN, D), lambda i: (i, 0)),
        ],
        out_shape=[
            jax.ShapeDtypeStruct((N_NODES, D), jnp.float32),
            jax.ShapeDtypeStruct((N_NODES, D), jnp.float32),
        ],
    )(s2, s2, g2, dinv, b2, wp1)


def _tc4_body(ga_ref, gb_ref, ea_ref, wp1b_ref, bp1_ref, wp2_ref, bp2_ref,
              out_ref):
    h = jax.nn.relu(ga_ref[...] + gb_ref[...]
                    + jnp.dot(ea_ref[...], wp1b_ref[...],
                              preferred_element_type=jnp.float32)
                    + bp1_ref[...][None, :])
    o = jnp.dot(h, wp2_ref[...], preferred_element_type=jnp.float32)
    out_ref[...] = o[:, 0] + bp2_ref[0]


def _tc4(ga, gb, ea, wp1b, bp1, wp2, bp2):
    return pl.pallas_call(
        _tc4_body,
        grid=(EPAD // BE,),
        in_specs=[
            pl.BlockSpec((BE, D), lambda i: (i, 0)),
            pl.BlockSpec((BE, D), lambda i: (i, 0)),
            pl.BlockSpec((BE, D_EDGE), lambda i: (i, 0)),
            pl.BlockSpec((D_EDGE, D), lambda i: (0, 0)),
            pl.BlockSpec((D,), lambda i: (0,)),
            pl.BlockSpec((D, 1), lambda i: (0, 0)),
            pl.BlockSpec((1,), lambda i: (0,)),
        ],
        out_specs=pl.BlockSpec((BE,), lambda i: (i,)),
        out_shape=jax.ShapeDtypeStruct((EPAD,), jnp.float32),
    )(ga, gb, ea, wp1b, bp1, wp2, bp2)


def kernel(x, edge_index, edge_attr, W1, b1, W2, b2, Wp1, bp1, Wp2, bp2):
    ei = edge_index.astype(jnp.int32)
    src, dst = ei[0], ei[1]
    npad = EPAD - N_EDGES
    zpad = jnp.zeros((npad,), jnp.int32)
    src_g = jnp.concatenate([src, zpad]).reshape(NW, NCH, CH)
    dst_g = jnp.concatenate([dst, zpad]).reshape(NW, NCH, CH)
    dst_s = jnp.concatenate([dst, jnp.full((npad,), N_NODES, jnp.int32)]
                            ).reshape(NW, NCH, CH)
    ea_pad = jnp.concatenate(
        [edge_attr, jnp.zeros((npad, D_EDGE), jnp.float32)])

    ones_h = jnp.ones((CH, D), jnp.float32)
    z128 = jnp.zeros((ZROWS, D), jnp.float32)

    dpart = _deg_call(dst_s, ones_h, z128)
    g1, dinv = _tc1(x, W1, dpart)
    s1 = _agg_call(g1, src_g, dst_s, z128)
    g2 = _tc2(s1, g1, dinv, b1, W2)
    s2 = _agg_call(g2, src_g, dst_s, z128)
    a, b = _tc3(s2, g2, dinv, b2, Wp1)
    ga, gb = _egather_call(a, b, src_g, dst_g)
    out = _tc4(ga, gb, ea_pad, Wp1[2 * D:, :], bp1, Wp2, bp2)
    return out[:N_EDGES]
